# Initial kernel scaffold; baseline (speedup 1.0000x reference)
import jax
import jax.numpy as jnp
from jax.experimental import pallas as pl


def kernel(x, seg, tok_embed, pos_embed, seg_embed, gamma, beta):
    raise NotImplementedError("write your pallas kernel here")


# speedup vs baseline: 3.8968x; 3.8968x over previous
import functools

import jax
import jax.numpy as jnp
from jax import lax
from jax.experimental import pallas as pl
from jax.experimental.pallas import tpu as pltpu
from jax.experimental.pallas import tpu_sc as plsc

_NUM_CORES = 2
_NUM_SUBCORES = 16
_NW = _NUM_CORES * _NUM_SUBCORES
_LANES = 16


def _table_body(tok_ref, pos_ref, seg_ref, gamma_ref, beta_ref, out_ref):
    V, D = tok_ref.shape
    M = pos_ref.shape[0]
    G = seg_ref.shape[0]
    e = (tok_ref[:][:, None, None, :]
         + pos_ref[:][None, :, None, :]
         + seg_ref[:][None, None, :, :])
    e = e.reshape(V * M * G, D)
    mean = jnp.mean(e, axis=-1, keepdims=True)
    c = e - mean
    var = jnp.mean(c * c, axis=-1, keepdims=True)
    out_ref[:] = c * lax.rsqrt(var + 1e-5) * gamma_ref[:] + beta_ref[:]


def _build_table(tok_embed, pos_embed, seg_embed, gamma, beta):
    V, D = tok_embed.shape
    M = pos_embed.shape[0]
    G = seg_embed.shape[0]
    return pl.pallas_call(
        _table_body,
        out_shape=jax.ShapeDtypeStruct((V * M * G, D), jnp.float32),
    )(tok_embed, pos_embed, seg_embed, gamma.reshape(1, D), beta.reshape(1, D))


def _make_sc_gather(B, D, M, G, n_chunk):
    b_per_w = B // _NW
    n_iters = b_per_w // n_chunk
    mesh = plsc.VectorSubcoreMesh(core_axis_name="c", subcore_axis_name="s")

    @functools.partial(
        pl.kernel,
        mesh=mesh,
        out_type=jax.ShapeDtypeStruct((B, D), jnp.float32),
        scratch_types=[
            pltpu.VMEM((b_per_w,), jnp.int32),
            pltpu.VMEM((b_per_w,), jnp.int32),
            pltpu.VMEM((n_chunk,), jnp.int32),
            pltpu.VMEM((n_chunk, D), jnp.float32),
            pltpu.SemaphoreType.DMA,
        ],
    )
    def sc_gather(x_hbm, seg_hbm, table_hbm, out_hbm, x_v, seg_v, idx_v,
                  rows_v, sem):
        wid = lax.axis_index("s") * _NUM_CORES + lax.axis_index("c")
        base = wid * b_per_w
        pltpu.sync_copy(x_hbm.at[pl.ds(base, b_per_w)], x_v)
        pltpu.sync_copy(seg_hbm.at[pl.ds(base, b_per_w)], seg_v)

        def chunk_body(k, _):
            off = k * n_chunk
            for c in range(n_chunk // _LANES):
                lane = lax.broadcasted_iota(jnp.int32, (_LANES,), 0)
                j = base + off + c * _LANES + lane
                p = lax.rem(j, M)
                xx = x_v[pl.ds(off + c * _LANES, _LANES)]
                ss = seg_v[pl.ds(off + c * _LANES, _LANES)]
                idx_v[pl.ds(c * _LANES, _LANES)] = xx * (M * G) + p * G + ss
            pltpu.async_copy(table_hbm.at[idx_v], rows_v, sem).wait()
            pltpu.sync_copy(rows_v, out_hbm.at[pl.ds(base + off, n_chunk)])
            return 0

        lax.fori_loop(0, n_iters, chunk_body, 0)

    return sc_gather


def kernel(x, seg, tok_embed, pos_embed, seg_embed, gamma, beta):
    Bt, S = x.shape
    V, D = tok_embed.shape
    M = pos_embed.shape[0]
    G = seg_embed.shape[0]
    B = Bt * S

    table = _build_table(tok_embed, pos_embed, seg_embed, gamma, beta)
    x_flat = x.reshape(B).astype(jnp.int32)
    seg_flat = seg.reshape(B).astype(jnp.int32)
    out_flat = _make_sc_gather(B, D, M, G, n_chunk=64)(x_flat, seg_flat, table)
    return out_flat.reshape(Bt, S, D)


# speedup vs baseline: 3.9103x; 1.0035x over previous
import functools

import jax
import jax.numpy as jnp
from jax import lax
from jax.experimental import pallas as pl
from jax.experimental.pallas import tpu as pltpu
from jax.experimental.pallas import tpu_sc as plsc

_NUM_CORES = 2
_NUM_SUBCORES = 16
_NW = _NUM_CORES * _NUM_SUBCORES
_LANES = 16


def _table_body(tok_ref, pos_ref, seg_ref, gamma_ref, beta_ref, out_ref):
    V, D = tok_ref.shape
    M = pos_ref.shape[0]
    G = seg_ref.shape[0]
    e = (tok_ref[:][:, None, None, :]
         + pos_ref[:][None, :, None, :]
         + seg_ref[:][None, None, :, :])
    e = e.reshape(V * M * G, D)
    mean = jnp.mean(e, axis=-1, keepdims=True)
    c = e - mean
    var = jnp.mean(c * c, axis=-1, keepdims=True)
    out_ref[:] = c * lax.rsqrt(var + 1e-5) * gamma_ref[:] + beta_ref[:]


def _build_table(tok_embed, pos_embed, seg_embed, gamma, beta):
    V, D = tok_embed.shape
    M = pos_embed.shape[0]
    G = seg_embed.shape[0]
    return pl.pallas_call(
        _table_body,
        out_shape=jax.ShapeDtypeStruct((V * M * G, D), jnp.float32),
    )(tok_embed, pos_embed, seg_embed, gamma.reshape(1, D), beta.reshape(1, D))


def _make_sc_gather(B, D, M, G, n_chunk):
    b_per_w = B // _NW
    n_iters = b_per_w // n_chunk
    n_pairs = n_iters // 2
    mesh = plsc.VectorSubcoreMesh(core_axis_name="c", subcore_axis_name="s")

    @functools.partial(
        pl.kernel,
        mesh=mesh,
        out_type=jax.ShapeDtypeStruct((B, D), jnp.float32),
        scratch_types=[
            pltpu.VMEM((b_per_w,), jnp.int32),
            pltpu.VMEM((b_per_w,), jnp.int32),
            pltpu.VMEM((b_per_w,), jnp.int32),
            pltpu.VMEM((n_chunk, D), jnp.float32),
            pltpu.VMEM((n_chunk, D), jnp.float32),
            pltpu.SemaphoreType.DMA,
            pltpu.SemaphoreType.DMA,
            pltpu.SemaphoreType.DMA,
            pltpu.SemaphoreType.DMA,
        ],
    )
    def sc_gather(x_hbm, seg_hbm, table_hbm, out_hbm, x_v, seg_v, idx_v,
                  rows0_v, rows1_v, g0, g1, s0, s1):
        wid = lax.axis_index("s") * _NUM_CORES + lax.axis_index("c")
        base = wid * b_per_w
        pltpu.sync_copy(x_hbm.at[pl.ds(base, b_per_w)], x_v)
        pltpu.sync_copy(seg_hbm.at[pl.ds(base, b_per_w)], seg_v)

        def idx_body(i, _):
            lane = lax.broadcasted_iota(jnp.int32, (_LANES,), 0)
            j = base + i * _LANES + lane
            p = lax.rem(j, M)
            xx = x_v[pl.ds(i * _LANES, _LANES)]
            ss = seg_v[pl.ds(i * _LANES, _LANES)]
            idx_v[pl.ds(i * _LANES, _LANES)] = xx * (M * G) + p * G + ss
            return 0

        lax.fori_loop(0, b_per_w // _LANES, idx_body, 0)

        def g_start(k, rbuf, sem):
            pltpu.async_copy(table_hbm.at[idx_v.at[pl.ds(k * n_chunk, n_chunk)]],
                             rbuf, sem)

        def g_wait(rbuf, sem):
            pltpu.make_async_copy(
                table_hbm.at[idx_v.at[pl.ds(0, n_chunk)]], rbuf, sem).wait()

        def s_start(k, rbuf, sem):
            pltpu.async_copy(rbuf, out_hbm.at[pl.ds(base + k * n_chunk, n_chunk)],
                             sem)

        def s_wait(rbuf, sem):
            pltpu.make_async_copy(
                rbuf, out_hbm.at[pl.ds(base, n_chunk)], sem).wait()

        g_start(0, rows0_v, g0)
        g_start(1, rows1_v, g1)

        def pair_body(i, _):
            a = 2 * i
            g_wait(rows0_v, g0)
            s_start(a, rows0_v, s0)
            g_wait(rows1_v, g1)
            s_start(a + 1, rows1_v, s1)

            @pl.when(i < n_pairs - 1)
            def _():
                s_wait(rows0_v, s0)
                g_start(a + 2, rows0_v, g0)
                s_wait(rows1_v, s1)
                g_start(a + 3, rows1_v, g1)

            return 0

        lax.fori_loop(0, n_pairs, pair_body, 0)
        s_wait(rows0_v, s0)
        s_wait(rows1_v, s1)

    return sc_gather


def kernel(x, seg, tok_embed, pos_embed, seg_embed, gamma, beta):
    Bt, S = x.shape
    V, D = tok_embed.shape
    M = pos_embed.shape[0]
    G = seg_embed.shape[0]
    B = Bt * S

    table = _build_table(tok_embed, pos_embed, seg_embed, gamma, beta)
    x_flat = x.reshape(B).astype(jnp.int32)
    seg_flat = seg.reshape(B).astype(jnp.int32)
    out_flat = _make_sc_gather(B, D, M, G, n_chunk=64)(x_flat, seg_flat, table)
    return out_flat.reshape(Bt, S, D)
